# Initial kernel scaffold; baseline (speedup 1.0000x reference)
import jax
import jax.numpy as jnp
from jax.experimental import pallas as pl


def kernel(node_feats, unary_src, binary_src, params):
    raise NotImplementedError("write your pallas kernel here")


# speedup vs baseline: 1.0557x; 1.0557x over previous
import functools

import jax
import jax.numpy as jnp
from jax import lax
from jax.experimental import pallas as pl
from jax.experimental.pallas import tpu as pltpu
from jax.experimental.pallas import tpu_sc as plsc

H = 128
N_NODES = 100000
NU_ = 50000
NB_ = 50000


def _embed_body(x_ref, w_ref, b_ref, o_ref):
    o_ref[...] = jnp.tanh(
        jnp.dot(x_ref[...], w_ref[...], preferred_element_type=jnp.float32)
        + b_ref[...]
    )


def _embed(x, w, b, blk):
    n = x.shape[0]
    return pl.pallas_call(
        _embed_body,
        grid=(n // blk,),
        in_specs=[
            pl.BlockSpec((blk, H), lambda i: (i, 0)),
            pl.BlockSpec((H, H), lambda i: (0, 0)),
            pl.BlockSpec((1, H), lambda i: (0, 0)),
        ],
        out_specs=pl.BlockSpec((blk, H), lambda i: (i, 0)),
        out_shape=jax.ShapeDtypeStruct((n, H), jnp.float32),
    )(x, w, b)


_NC = 2
_NS = 16
_NW = _NC * _NS
_CH = 128


def _sc_gather(table, idx3d, steps):
    total = _NW * steps * _CH
    mesh = plsc.VectorSubcoreMesh(core_axis_name="c", subcore_axis_name="s")

    @functools.partial(
        pl.kernel,
        mesh=mesh,
        out_type=jax.ShapeDtypeStruct((total, H), jnp.float32),
        scratch_types=[
            pltpu.VMEM((steps, _CH), jnp.int32),
            pltpu.VMEM((_CH, H), jnp.float32),
            pltpu.SemaphoreType.DMA,
        ],
    )
    def gather_kernel(table_hbm, idx_hbm, out_hbm, idx_v, rows_v, sem):
        wid = lax.axis_index("s") * _NC + lax.axis_index("c")
        row0 = wid * steps
        pltpu.sync_copy(idx_hbm.at[wid], idx_v)

        def step(j, carry):
            pltpu.async_copy(table_hbm.at[idx_v.at[j]], rows_v, sem).wait()
            pltpu.sync_copy(rows_v, out_hbm.at[pl.ds((row0 + j) * _CH, _CH)])
            return carry

        lax.fori_loop(0, steps, step, 0)

    return gather_kernel(table, idx3d)


def _chain_body(
    x_ref, emb_ref, w0_ref, b0_ref, w1_ref, b1_ref,
    wa_ref, wb_ref, bab_ref, wnx_ref, wne_ref, bn_ref, o_ref
):
    f32 = jnp.float32
    x = x_ref[...]
    r0 = jnp.tanh(jnp.dot(x, w0_ref[...], preferred_element_type=f32) + b0_ref[...])
    r = jnp.tanh(jnp.dot(r0, w1_ref[...], preferred_element_type=f32) + b1_ref[...])
    for i in range(4):
        r = jnp.tanh(
            jnp.dot(r, wa_ref[i], preferred_element_type=f32)
            + jnp.dot(r0, wb_ref[i], preferred_element_type=f32)
            + bab_ref[i]
        )
    emb = emb_ref[...]
    e = jnp.tanh(
        jnp.dot(emb, wne_ref[0], preferred_element_type=f32)
        + jnp.dot(r, wnx_ref[0], preferred_element_type=f32)
        + bn_ref[0]
    )
    for i in range(1, 5):
        e = jnp.tanh(
            jnp.dot(e, wnx_ref[i], preferred_element_type=f32)
            + jnp.dot(emb, wne_ref[i], preferred_element_type=f32)
            + bn_ref[i]
        )
    o_ref[...] = e


def _chain(x, emb, emb_blk_off, w0, b0, w1, b1, wa, wb, bab, wnx, wne, bn, blk):
    n = x.shape[0]
    d_in = x.shape[1]
    return pl.pallas_call(
        _chain_body,
        grid=(n // blk,),
        in_specs=[
            pl.BlockSpec((blk, d_in), lambda i: (i, 0)),
            pl.BlockSpec((blk, H), lambda i, o=emb_blk_off: (i + o, 0)),
            pl.BlockSpec((d_in, H), lambda i: (0, 0)),
            pl.BlockSpec((1, H), lambda i: (0, 0)),
            pl.BlockSpec((H, H), lambda i: (0, 0)),
            pl.BlockSpec((1, H), lambda i: (0, 0)),
            pl.BlockSpec((4, H, H), lambda i: (0, 0, 0)),
            pl.BlockSpec((4, H, H), lambda i: (0, 0, 0)),
            pl.BlockSpec((4, 1, H), lambda i: (0, 0, 0)),
            pl.BlockSpec((5, H, H), lambda i: (0, 0, 0)),
            pl.BlockSpec((5, H, H), lambda i: (0, 0, 0)),
            pl.BlockSpec((5, 1, H), lambda i: (0, 0, 0)),
        ],
        out_specs=pl.BlockSpec((blk, H), lambda i: (i, 0)),
        out_shape=jax.ShapeDtypeStruct((n, H), jnp.float32),
    )(x, emb, w0, b0, w1, b1, wa, wb, bab, wnx, wne, bn)


def kernel(node_feats, unary_src, binary_src, params):
    p = params
    blk = 1000

    emb = _embed(node_feats, p["We"], p["be"].reshape(1, H), blk)

    idx = jnp.concatenate([unary_src, binary_src.reshape(-1)])
    steps = 37
    pad_rows = _NW * steps * _CH
    idx = jnp.concatenate(
        [idx, jnp.zeros((pad_rows - idx.shape[0],), jnp.int32)]
    )
    g = _sc_gather(emb, idx.reshape(_NW, steps, _CH), steps)

    g2 = g.reshape(pad_rows // 2, 2 * H)

    def stk(names, sl):
        return jnp.stack([p[n][sl] for n in names])

    def stkb(names):
        return jnp.stack([p[n].reshape(1, H) for n in names])

    lo, hi = slice(0, H), slice(H, 2 * H)
    u_names = ["Wu%d" % i for i in range(2, 6)]
    b_names = ["Wb%d" % i for i in range(2, 6)]
    n_names = ["Wn%d" % i for i in range(5)]
    wa_u, wb_u = stk(u_names, lo), stk(u_names, hi)
    bab_u = stkb(["bu%d" % i for i in range(2, 6)])
    wa_b, wb_b = stk(b_names, lo), stk(b_names, hi)
    bab_b = stkb(["bb%d" % i for i in range(2, 6)])
    wnx = jnp.stack([p["Wn0"][hi]] + [p[n][lo] for n in n_names[1:]])
    wne = jnp.stack([p["Wn0"][lo]] + [p[n][hi] for n in n_names[1:]])
    bn = stkb(["bn%d" % i for i in range(5)])

    e_u = _chain(
        g[:NU_], emb, 0,
        p["Wu0"], p["bu0"].reshape(1, H), p["Wu1"], p["bu1"].reshape(1, H),
        wa_u, wb_u, bab_u, wnx, wne, bn, blk,
    )
    e_b = _chain(
        g2[NU_ // 2 : NU_ // 2 + NB_], emb, NU_ // blk,
        p["Wb0"], p["bb0"].reshape(1, H), p["Wb1"], p["bb1"].reshape(1, H),
        wa_b, wb_b, bab_b, wnx, wne, bn, blk,
    )
    return jnp.concatenate([e_u, e_b], axis=0)
